# baseline (speedup 1.0000x reference)
import functools

import jax
import jax.numpy as jnp
from jax import lax
from jax.experimental import pallas as pl
from jax.experimental.pallas import tpu as pltpu
from jax.experimental.pallas import tpu_sc as plsc

N_ENTITIES = 1000000
N_RELATIONS = 1000
EMBED_DIM = 64
BATCH = 16384

NC = 2
NS = 16
NW = NC * NS
NLANE = 16

TILES_E = N_ENTITIES // 8
TILES_R = N_RELATIONS // 8
ROWS_A = N_ENTITIES // 2
ROWS_AR = 512
T_CHUNK = 16
NT_BASE = TILES_E // NW
NT_EXTRA = TILES_E - NT_BASE * NW
NSTEP = 246

B_PER_W = BATCH // NW
CHUNK = 128
NCHUNK = B_PER_W // CHUNK


def _compact_kernel(ent_hbm, rel_hbm, out_e, out_r,
                    stgs, cbs, rstg, rcb, semr, semw):
    wid = lax.axis_index("s") * NC + lax.axis_index("c")
    ent_v = ent_hbm.reshape(TILES_E, 8, EMBED_DIM)
    rel_v = rel_hbm.reshape(TILES_R, 8, EMBED_DIM)

    t0 = wid * NT_BASE + 2 * jnp.minimum(wid, NT_EXTRA // 2)
    nt = NT_BASE + jnp.where(wid < NT_EXTRA // 2, 2, 0)
    t_last = t0 + nt - T_CHUNK

    def tb_of(step):
        return jnp.minimum(t0 + step * T_CHUNK, t_last)

    def fire_read(step, slot):
        return pltpu.async_copy(
            ent_v.at[pl.ds(tb_of(step), T_CHUNK)], stgs[slot], semr[slot])

    def wait_read(slot):
        pltpu.make_async_copy(ent_v.at[pl.ds(0, T_CHUNK)], stgs[slot],
                              semr[slot]).wait()

    def fire_write(step, slot):
        return pltpu.async_copy(
            cbs[slot], out_e.at[pl.ds(tb_of(step) * 4, T_CHUNK * 4)],
            semw[slot])

    def wait_write(slot):
        pltpu.make_async_copy(cbs[slot],
                              out_e.at[pl.ds(0, T_CHUNK * 4)],
                              semw[slot]).wait()

    def compact(slot):
        stg, cb = stgs[slot], cbs[slot]
        for s in range(T_CHUNK):
            for p in range(4):
                for cc in range(EMBED_DIM // NLANE):
                    sl = pl.ds(cc * NLANE, NLANE)
                    cb[s * 4 + p, sl] = stg[s, 2 * p, sl]
                    cb[s * 4 + p, pl.ds(EMBED_DIM + cc * NLANE, NLANE)] = (
                        stg[s, 2 * p + 1, sl])

    for i in range(4):
        tr = jnp.minimum(4 * wid + i, TILES_R - 1)
        pltpu.sync_copy(rel_v.at[tr], rstg)
        for p in range(4):
            for cc in range(EMBED_DIM // NLANE):
                sl = pl.ds(cc * NLANE, NLANE)
                rcb[4 * i + p, sl] = rstg[2 * p, sl]
                rcb[4 * i + p, pl.ds(EMBED_DIM + cc * NLANE, NLANE)] = (
                    rstg[2 * p + 1, sl])
    pltpu.sync_copy(rcb, out_r.at[pl.ds(wid * NLANE, NLANE)])

    fire_read(0, 0)
    fire_read(1, 1)
    for k in (0, 1):
        wait_read(k)
        compact(k)
        fire_write(k, k)
        fire_read(k + 2, k)

    def body(g, carry):
        for b in range(2):
            k = g * 2 + b
            wait_read(b)
            wait_write(b)
            compact(b)
            fire_write(k, b)
            fire_read(k + 2, b)
        return carry

    lax.fori_loop(1, NSTEP // 2, body, 0)
    for b in range(2):
        wait_read(b)
        wait_write(b)


def _score_kernel(heads_hbm, rels_hbm, tails_hbm, tab_e, tab_r, out_hbm,
                  hidx, ridx, tidx, hpr, rpr, tpr, hhf, rhf, thf,
                  ebufs, rbufs, tbufs, outb, sems):
    wid = lax.axis_index("s") * NC + lax.axis_index("c")
    base = wid * B_PER_W

    pltpu.sync_copy(heads_hbm.at[pl.ds(base, B_PER_W)], hidx)
    pltpu.sync_copy(rels_hbm.at[pl.ds(base, B_PER_W)], ridx)
    pltpu.sync_copy(tails_hbm.at[pl.ds(base, B_PER_W)], tidx)
    for g in range(B_PER_W // NLANE):
        sl = pl.ds(g * NLANE, NLANE)
        for raw, pr, hf in ((hidx, hpr, hhf), (ridx, rpr, rhf),
                            (tidx, tpr, thf)):
            v = raw[sl]
            pr[sl] = lax.shift_right_logical(v, 1)
            hf[sl] = lax.bitwise_and(v, 1) * EMBED_DIM

    lane = lax.iota(jnp.int32, NLANE)

    def fire(k, slot):
        kk = jnp.minimum(k, NCHUNK - 1)
        isl = pl.ds(kk * CHUNK, CHUNK)
        c1 = pltpu.async_copy(tab_e.at[hpr.at[isl]], ebufs[slot], sems[slot])
        c2 = pltpu.async_copy(tab_e.at[tpr.at[isl]], tbufs[slot], sems[slot])
        c3 = pltpu.async_copy(tab_r.at[rpr.at[isl]], rbufs[slot], sems[slot])
        return (c1, c2, c3)

    def wait(slot):
        for buf in (ebufs, tbufs, rbufs):
            pltpu.make_async_copy(tab_e.at[hpr.at[pl.ds(0, CHUNK)]],
                                  buf[slot], sems[slot]).wait()

    def compute(k, slot):
        eb, rb, tb = ebufs[slot], rbufs[slot], tbufs[slot]

        def group_body(g, carry):
            gsl = pl.ds(k * CHUNK + g * NLANE, NLANE)
            hhv, rhv, thv = hhf[gsl], rhf[gsl], thf[gsl]
            out16 = jnp.zeros((NLANE,), jnp.float32)
            for i in range(NLANE):
                j = g * NLANE + i
                bh, br, bt = hhv[i], rhv[i], thv[i]
                acc = None
                for c in range(EMBED_DIM // NLANE):
                    off = c * NLANE
                    v = (eb[j, pl.ds(bh + off, NLANE)]
                         + rb[j, pl.ds(br + off, NLANE)]
                         - tb[j, pl.ds(bt + off, NLANE)])
                    acc = v * v if acc is None else acc + v * v
                out16 = jnp.where(lane == i, -jnp.sum(acc), out16)
            outb[pl.ds(k * CHUNK + g * NLANE, NLANE)] = out16
            return carry

        lax.fori_loop(0, CHUNK // NLANE, group_body, 0)

    fire(0, 0)

    def body(g, carry):
        for b in range(2):
            k = g * 2 + b
            fire(k + 1, 1 - b)
            wait(b)
            compute(k, b)
        return carry

    lax.fori_loop(0, NCHUNK // 2, body, 0)
    wait(0)

    pltpu.sync_copy(outb, out_hbm.at[pl.ds(base, B_PER_W)])


_MESH = plsc.VectorSubcoreMesh(core_axis_name="c", subcore_axis_name="s")
_PARAMS = pltpu.CompilerParams(needs_layout_passes=False,
                               use_tc_tiling_on_sc=True)


@jax.jit
def kernel(heads, relations, tails, entity_embedding, relation_embedding):
    compact = functools.partial(
        pl.kernel,
        mesh=_MESH,
        out_type=(jax.ShapeDtypeStruct((ROWS_A, 2 * EMBED_DIM), jnp.float32),
                  jax.ShapeDtypeStruct((ROWS_AR, 2 * EMBED_DIM), jnp.float32)),
        compiler_params=_PARAMS,
        scratch_types=[
            [pltpu.VMEM((T_CHUNK, 8, EMBED_DIM), jnp.float32) for _ in range(2)],
            [pltpu.VMEM((T_CHUNK * 4, 2 * EMBED_DIM), jnp.float32) for _ in range(2)],
            pltpu.VMEM((8, EMBED_DIM), jnp.float32),
            pltpu.VMEM((NLANE, 2 * EMBED_DIM), jnp.float32),
            [pltpu.SemaphoreType.DMA for _ in range(2)],
            [pltpu.SemaphoreType.DMA for _ in range(2)],
        ],
    )(_compact_kernel)
    tab_e, tab_r = compact(entity_embedding, relation_embedding)

    score = functools.partial(
        pl.kernel,
        mesh=_MESH,
        out_type=jax.ShapeDtypeStruct((BATCH,), jnp.float32),
        compiler_params=_PARAMS,
        scratch_types=[
            pltpu.VMEM((B_PER_W,), jnp.int32),
            pltpu.VMEM((B_PER_W,), jnp.int32),
            pltpu.VMEM((B_PER_W,), jnp.int32),
            pltpu.VMEM((B_PER_W,), jnp.int32),
            pltpu.VMEM((B_PER_W,), jnp.int32),
            pltpu.VMEM((B_PER_W,), jnp.int32),
            pltpu.VMEM((B_PER_W,), jnp.int32),
            pltpu.VMEM((B_PER_W,), jnp.int32),
            pltpu.VMEM((B_PER_W,), jnp.int32),
            [pltpu.VMEM((CHUNK, 2 * EMBED_DIM), jnp.float32) for _ in range(2)],
            [pltpu.VMEM((CHUNK, 2 * EMBED_DIM), jnp.float32) for _ in range(2)],
            [pltpu.VMEM((CHUNK, 2 * EMBED_DIM), jnp.float32) for _ in range(2)],
            pltpu.VMEM((B_PER_W,), jnp.float32),
            [pltpu.SemaphoreType.DMA for _ in range(2)],
        ],
    )(_score_kernel)
    return score(heads, relations, tails, tab_e, tab_r)
